# baseline (speedup 1.0000x reference)
import functools

import jax
import jax.numpy as jnp
from jax import lax
from jax.experimental import pallas as pl
from jax.experimental.pallas import tpu as pltpu
from jax.experimental.pallas import tpu_sc as plsc

TOPK = 8
NE = 64
NG = 8
EPG = NE // NG
TOPG = 4
SCALE_C = 2.5
L = 16
NW = 32
T = 32768
TPW = T // NW
NB = TPW // L


def _argmax_tree(vals, idxs):
    while len(vals) > 1:
        nv, ni = [], []
        for j in range(0, len(vals) - 1, 2):
            c = vals[j] >= vals[j + 1]
            nv.append(jnp.where(c, vals[j], vals[j + 1]))
            ni.append(jnp.where(c, idxs[j], idxs[j + 1]))
        if len(vals) % 2:
            nv.append(vals[-1])
            ni.append(idxs[-1])
        vals, idxs = nv, ni
    return vals[0], idxs[0]


def _recip(d):
    bits = plsc.bitcast(d, jnp.int32)
    r = plsc.bitcast(jnp.int32(0x7EF311C3) - bits, jnp.float32)
    for _ in range(3):
        r = r * (2.0 - d * r)
    return r


def _router_body(logits_hbm, bias_hbm, out_w_hbm, out_i_hbm,
                 logits_v, bias_v, s_scr, b_scr, m_scr,
                 out_w_v, out_i_v):
    wid = lax.axis_index("s") * 2 + lax.axis_index("c")
    base = wid * TPW
    pltpu.sync_copy(logits_hbm.at[pl.ds(base * NE, TPW * NE)], logits_v)
    pltpu.sync_copy(bias_hbm, bias_v)

    lane = lax.iota(jnp.int32, L)
    neg_inf = jnp.full((L,), -jnp.inf, jnp.float32)
    zero_i = jnp.zeros((L,), jnp.int32)

    @pl.loop(0, NB)
    def _batch(bi):
        tok = bi * L + lane
        tok_e = tok * NE
        tok_k = tok * TOPK

        with jax.named_scope("prepass"):
            m1 = [neg_inf] * NG
            m2 = [neg_inf] * NG
            for e in range(NE):
                x = plsc.load_gather(logits_v, [tok_e + jnp.int32(e)])
                s = _recip(1.0 + jnp.exp(-x))
                b = s + bias_v[pl.ds(e * L, L)]
                s_scr[pl.ds(e * L, L)] = s
                b_scr[pl.ds(e * L, L)] = b
                g = e // EPG
                lo = jnp.minimum(m1[g], b)
                m1[g] = jnp.maximum(m1[g], b)
                m2[g] = jnp.maximum(m2[g], lo)

        with jax.named_scope("grpsel"):
            gv = [m1[g] + m2[g] for g in range(NG)]
            for _ in range(TOPG):
                _, bg = _argmax_tree(list(gv),
                                     [jnp.int32(g) for g in range(NG)])
                for g in range(NG):
                    gv[g] = jnp.where(bg == jnp.int32(g), neg_inf, gv[g])
            sel = [gv[g] == neg_inf for g in range(NG)]

            for e in range(NE):
                m_scr[pl.ds(e * L, L)] = jnp.where(sel[e // EPG],
                                                   b_scr[pl.ds(e * L, L)],
                                                   neg_inf)

        with jax.named_scope("extract"):
            ssum = jnp.zeros((L,), jnp.float32)
            svals = []
            for k in range(TOPK):
                _, bix = _argmax_tree(
                    [m_scr[pl.ds(e * L, L)] for e in range(NE)],
                    [jnp.int32(e) for e in range(NE)])
                flat = bix * L + lane
                plsc.store_scatter(m_scr, [flat], neg_inf)
                sv = plsc.load_gather(s_scr, [flat])
                svals.append(sv)
                ssum = ssum + sv
                plsc.store_scatter(out_i_v, [tok_k + jnp.int32(k)], bix)

        with jax.named_scope("norm"):
            inv = SCALE_C / (ssum + 1e-20)
            for k in range(TOPK):
                plsc.store_scatter(out_w_v, [tok_k + jnp.int32(k)],
                                   svals[k] * inv)

    pltpu.sync_copy(out_w_v, out_w_hbm.at[pl.ds(base * TOPK, TPW * TOPK)])
    pltpu.sync_copy(out_i_v, out_i_hbm.at[pl.ds(base * TOPK, TPW * TOPK)])


@jax.jit
def _route(router_logits, expert_bias):
    mesh = plsc.VectorSubcoreMesh(core_axis_name="c", subcore_axis_name="s")
    run = functools.partial(
        pl.kernel,
        out_type=(
            jax.ShapeDtypeStruct((T * TOPK,), jnp.float32),
            jax.ShapeDtypeStruct((T * TOPK,), jnp.int32),
        ),
        mesh=mesh,
        compiler_params=pltpu.CompilerParams(needs_layout_passes=False),
        scratch_types=[
            pltpu.VMEM((TPW * NE,), jnp.float32),
            pltpu.VMEM((NE * L,), jnp.float32),
            pltpu.VMEM((NE * L,), jnp.float32),
            pltpu.VMEM((NE * L,), jnp.float32),
            pltpu.VMEM((NE * L,), jnp.float32),
            pltpu.VMEM((TPW * TOPK,), jnp.float32),
            pltpu.VMEM((TPW * TOPK,), jnp.int32),
        ],
    )(_router_body)
    bias_rep = jnp.broadcast_to(expert_bias[:, None], (NE, L)).reshape(-1)
    w, i = run(router_logits.reshape(-1), bias_rep)
    return w.reshape(T, TOPK), i.reshape(T, TOPK)


def kernel(router_logits, expert_bias):
    w, i = _route(router_logits, expert_bias)
    return (w, i, router_logits)


# speedup vs baseline: 1.2077x; 1.2077x over previous
import functools

import jax
import jax.numpy as jnp
from jax import lax
from jax.experimental import pallas as pl
from jax.experimental.pallas import tpu as pltpu
from jax.experimental.pallas import tpu_sc as plsc

TOPK = 8
NE = 64
NG = 8
EPG = NE // NG
TOPG = 4
SCALE_C = 2.5
L = 16
NW = 32
T = 32768
TPW = T // NW
NB = TPW // L
NEP = NE + 1


def _argmax_tree(vals, idxs):
    while len(vals) > 1:
        nv, ni = [], []
        for j in range(0, len(vals) - 1, 2):
            c = vals[j] >= vals[j + 1]
            nv.append(jnp.where(c, vals[j], vals[j + 1]))
            ni.append(jnp.where(c, idxs[j], idxs[j + 1]))
        if len(vals) % 2:
            nv.append(vals[-1])
            ni.append(idxs[-1])
        vals, idxs = nv, ni
    return vals[0], idxs[0]


def _recip(d):
    bits = plsc.bitcast(d, jnp.int32)
    r = plsc.bitcast(jnp.int32(0x7EF311C3) - bits, jnp.float32)
    for _ in range(3):
        r = r * (2.0 - d * r)
    return r


def _router_body(logits_hbm, bias_hbm, out_w_hbm, out_i_hbm,
                 logits_v, bias_v, s_scr, b_scr, m_scr,
                 out_w_v, out_i_v):
    wid = lax.axis_index("s") * 2 + lax.axis_index("c")
    base = wid * TPW
    pltpu.sync_copy(logits_hbm.at[pl.ds(base * NEP, TPW * NEP)], logits_v)
    pltpu.sync_copy(bias_hbm, bias_v)

    lane = lax.iota(jnp.int32, L)
    neg_inf = jnp.full((L,), -jnp.inf, jnp.float32)
    zero_i = jnp.zeros((L,), jnp.int32)

    @pl.loop(0, NB)
    def _batch(bi):
        tok = bi * L + lane
        tok_e = tok * NEP
        tok_k = tok * TOPK

        with jax.named_scope("prepass"):
            m1 = [neg_inf] * NG
            m2 = [neg_inf] * NG
            for e in range(NE):
                x = plsc.load_gather(logits_v, [tok_e + jnp.int32(e)])
                s = 1.0 / (1.0 + jnp.exp(-x))
                b = s + bias_v[pl.ds(e * L, L)]
                s_scr[pl.ds(e * L, L)] = s
                b_scr[pl.ds(e * L, L)] = b
                g = e // EPG
                lo = jnp.minimum(m1[g], b)
                m1[g] = jnp.maximum(m1[g], b)
                m2[g] = jnp.maximum(m2[g], lo)

        with jax.named_scope("grpsel"):
            gv = [m1[g] + m2[g] for g in range(NG)]
            for _ in range(TOPG):
                _, bg = _argmax_tree(list(gv),
                                     [jnp.int32(g) for g in range(NG)])
                for g in range(NG):
                    gv[g] = jnp.where(bg == jnp.int32(g), neg_inf, gv[g])
            sel = [gv[g] == neg_inf for g in range(NG)]

            for e in range(NE):
                m_scr[pl.ds(e * L, L)] = jnp.where(sel[e // EPG],
                                                   b_scr[pl.ds(e * L, L)],
                                                   neg_inf)

        with jax.named_scope("extract"):
            ssum = jnp.zeros((L,), jnp.float32)
            svals = []
            for k in range(TOPK):
                _, bix = _argmax_tree(
                    [m_scr[pl.ds(e * L, L)] for e in range(NE)],
                    [jnp.int32(e) for e in range(NE)])
                flat = bix * L + lane
                plsc.store_scatter(m_scr, [flat], neg_inf)
                sv = plsc.load_gather(s_scr, [flat])
                svals.append(sv)
                ssum = ssum + sv
                plsc.store_scatter(out_i_v, [tok_k + jnp.int32(k)], bix)

        with jax.named_scope("norm"):
            inv = SCALE_C / (ssum + 1e-20)
            for k in range(TOPK):
                plsc.store_scatter(out_w_v, [tok_k + jnp.int32(k)],
                                   svals[k] * inv)

    pltpu.sync_copy(out_w_v, out_w_hbm.at[pl.ds(base * TOPK, TPW * TOPK)])
    pltpu.sync_copy(out_i_v, out_i_hbm.at[pl.ds(base * TOPK, TPW * TOPK)])


@jax.jit
def _route(router_logits, expert_bias):
    mesh = plsc.VectorSubcoreMesh(core_axis_name="c", subcore_axis_name="s")
    run = functools.partial(
        pl.kernel,
        out_type=(
            jax.ShapeDtypeStruct((T * TOPK,), jnp.float32),
            jax.ShapeDtypeStruct((T * TOPK,), jnp.int32),
        ),
        mesh=mesh,
        compiler_params=pltpu.CompilerParams(needs_layout_passes=False),
        scratch_types=[
            pltpu.VMEM((TPW * NEP,), jnp.float32),
            pltpu.VMEM((NE * L,), jnp.float32),
            pltpu.VMEM((NE * L,), jnp.float32),
            pltpu.VMEM((NE * L,), jnp.float32),
            pltpu.VMEM((NE * L,), jnp.float32),
            pltpu.VMEM((TPW * TOPK,), jnp.float32),
            pltpu.VMEM((TPW * TOPK,), jnp.int32),
        ],
    )(_router_body)
    bias_rep = jnp.broadcast_to(expert_bias[:, None], (NE, L)).reshape(-1)
    logits_pad = jnp.pad(router_logits, ((0, 0), (0, NEP - NE)))
    w, i = run(logits_pad.reshape(-1), bias_rep)
    return w.reshape(T, TOPK), i.reshape(T, TOPK)


def kernel(router_logits, expert_bias):
    w, i = _route(router_logits, expert_bias)
    return (w, i, router_logits)
